# baseline (speedup 1.0000x reference)
import functools

import jax
import jax.numpy as jnp
from jax import lax
from jax.experimental import pallas as pl
from jax.experimental.pallas import tpu as pltpu
from jax.experimental.pallas import tpu_sc as plsc

B = 16384
D = 16
NEG = 4
K = NEG + 1
NC = 2
NS = 16
NW = NC * NS
BPW = B // NW
CPW = (K * B) // NW

_mesh = plsc.VectorSubcoreMesh(
    core_axis_name="c", subcore_axis_name="s", num_cores=NC, num_subcores=NS
)


@functools.partial(
    pl.kernel,
    out_type=(
        jax.ShapeDtypeStruct((B, D), jnp.float32),
        jax.ShapeDtypeStruct((K * B, D), jnp.float32),
    ),
    mesh=_mesh,
    compiler_params=pltpu.CompilerParams(use_tc_tiling_on_sc=False),
    scratch_types=[
        pltpu.VMEM((BPW,), jnp.int32),
        pltpu.VMEM((CPW,), jnp.int32),
        pltpu.VMEM((BPW, D), jnp.float32),
        pltpu.VMEM((CPW, D), jnp.float32),
        pltpu.SemaphoreType.DMA,
        pltpu.SemaphoreType.DMA,
    ],
)
def _sc_gather(uid_hbm, cidx_hbm, ut_hbm, it_hbm, ue_out, cat_out,
               idx_u, idx_c, rows_u, rows_c, s1, s2):
    wid = lax.axis_index("c") * NS + lax.axis_index("s")
    ub = wid * BPW
    cb = wid * CPW
    pltpu.sync_copy(uid_hbm.at[pl.ds(ub, BPW)], idx_u)
    pltpu.sync_copy(cidx_hbm.at[pl.ds(cb, CPW)], idx_c)
    cu = pltpu.async_copy(ut_hbm.at[idx_u], rows_u, s1)
    cc = pltpu.async_copy(it_hbm.at[idx_c], rows_c, s2)
    cu.wait()
    cc.wait()
    pltpu.sync_copy(rows_u, ue_out.at[pl.ds(ub, BPW)])
    pltpu.sync_copy(rows_c, cat_out.at[pl.ds(cb, CPW)])


BLK = 2048


def _tc_loss_body(ue_ref, cat_ref, out_ref):
    u = ue_ref[...]
    su = jnp.sum(u * u, axis=1, keepdims=True)
    ru = jnp.sqrt(su)
    acc = None
    for k in range(K):
        c = cat_ref[k]
        dot = jnp.sum(u * c, axis=1, keepdims=True)
        sc2 = jnp.sum(c * c, axis=1, keepdims=True)
        denom = jnp.sqrt(sc2) * ru + 1e-6
        d = (dot / denom + 1.0) * 0.5
        term = jnp.log(d + 1e-6) if k == 0 else jnp.log(1.0 - d + 1e-6)
        acc = term if acc is None else acc + term

    @pl.when(pl.program_id(0) == 0)
    def _():
        out_ref[0, 0] = 0.0

    out_ref[0, 0] += -jnp.sum(acc) / B


_tc_loss = pl.pallas_call(
    _tc_loss_body,
    grid=(B // BLK,),
    in_specs=[
        pl.BlockSpec((BLK, D), lambda i: (i, 0)),
        pl.BlockSpec((K, BLK, D), lambda i: (0, i, 0)),
    ],
    out_shape=jax.ShapeDtypeStruct((1, 1), jnp.float32),
    out_specs=pl.BlockSpec(memory_space=pltpu.SMEM),
)


def kernel(userid, itemid, user_feature, item_feature, neg_sample,
           user_table, item_table):
    uid = userid.reshape(B).astype(jnp.int32)
    cat_idx = jnp.concatenate(
        [itemid.astype(jnp.int32), neg_sample.astype(jnp.int32)], axis=1
    ).T.reshape(K * B)
    ue, cat = _sc_gather(uid, cat_idx, user_table, item_table)
    return _tc_loss(ue, cat.reshape(K, B, D))[0, 0]
